# baseline (speedup 1.0000x reference)
import jax
import jax.numpy as jnp
from jax.experimental import pallas as pl
from jax.experimental.pallas import tpu as pltpu

_DIM = 384
_EPS = 1e-10
_B = 32
_HW = 1024
_BB = 2


def _vq_kernel(x_ref, quant_ref, embed_ref, perp_ref, div_ref, hist_ref, qsum_ref):
    i = pl.program_id(0)
    nsteps = pl.num_programs(0)

    hist_part = jnp.zeros((_DIM, 1), jnp.float32)
    qsum_part = jnp.zeros((_DIM, 1), jnp.float32)
    iota = jax.lax.broadcasted_iota(jnp.int32, (_DIM, _HW), 0)

    for k in range(_BB):
        xb = x_ref[k]
        xr = jnp.maximum(xb, 0.0)
        s = jnp.sum(xr, axis=0, keepdims=True)
        r = 1.0 / (s + _EPS)
        xn = xr * r

        m = jnp.max(xr, axis=0, keepdims=True)
        inds = jnp.min(jnp.where(xr == m, iota, _DIM), axis=0, keepdims=True)

        one_hot = (iota == inds).astype(jnp.float32)
        quant_ref[k] = one_hot
        embed_ref[k] = inds

        hist_part += jnp.sum(one_hot, axis=1, keepdims=True)
        qsum_part += jnp.sum(xn, axis=1, keepdims=True)

    @pl.when(i == 0)
    def _init():
        hist_ref[...] = hist_part
        qsum_ref[...] = qsum_part

    @pl.when(i > 0)
    def _acc():
        hist_ref[...] += hist_part
        qsum_ref[...] += qsum_part

    @pl.when(i == nsteps - 1)
    def _finalize():
        n = float(_B * _HW)
        avg_probs = hist_ref[...] / n
        ent = jnp.sum(avg_probs * jnp.log(avg_probs + 1e-10), axis=0, keepdims=True)
        perp_ref[...] = jnp.exp(-ent)
        q_bar = qsum_ref[...] / n
        div_ref[...] = jnp.mean((q_bar * float(_DIM) - 1.0) ** 2, axis=0, keepdims=True)


def kernel(x):
    b, dim, h, w = x.shape
    hw = h * w
    xr = x.reshape(b, dim, hw)

    quant, embed, perp, div, _hist, _qsum = pl.pallas_call(
        _vq_kernel,
        grid=(b // _BB,),
        in_specs=[pl.BlockSpec((_BB, dim, hw), lambda i: (i, 0, 0))],
        out_specs=[
            pl.BlockSpec((_BB, dim, hw), lambda i: (i, 0, 0)),
            pl.BlockSpec((_BB, 1, hw), lambda i: (i, 0, 0)),
            pl.BlockSpec((1, 1), lambda i: (0, 0)),
            pl.BlockSpec((1, 1), lambda i: (0, 0)),
            pl.BlockSpec((dim, 1), lambda i: (0, 0)),
            pl.BlockSpec((dim, 1), lambda i: (0, 0)),
        ],
        out_shape=[
            jax.ShapeDtypeStruct((b, dim, hw), jnp.float32),
            jax.ShapeDtypeStruct((b, 1, hw), jnp.int32),
            jax.ShapeDtypeStruct((1, 1), jnp.float32),
            jax.ShapeDtypeStruct((1, 1), jnp.float32),
            jax.ShapeDtypeStruct((dim, 1), jnp.float32),
            jax.ShapeDtypeStruct((dim, 1), jnp.float32),
        ],
    )(xr)

    quantize = quant.reshape(b, dim, h, w)
    embed_ind = embed.reshape(b, h, w)
    return (quantize, div[0, 0], embed_ind, perp[0, 0])


# speedup vs baseline: 1.0285x; 1.0285x over previous
import jax
import jax.numpy as jnp
from jax.experimental import pallas as pl
from jax.experimental.pallas import tpu as pltpu

_DIM = 384
_EPS = 1e-10
_B = 32
_HW = 1024
_BB = 4


def _vq_kernel(x_ref, quant_ref, embed_ref, perp_ref, div_ref, hist_ref, qsum_ref):
    i = pl.program_id(0)
    nsteps = pl.num_programs(0)

    hist_part = jnp.zeros((_DIM, 1), jnp.float32)
    qsum_part = jnp.zeros((_DIM, 1), jnp.float32)
    iota = jax.lax.broadcasted_iota(jnp.int32, (_DIM, _HW), 0)

    for k in range(_BB):
        xb = x_ref[k]
        xr = jnp.maximum(xb, 0.0)
        s = jnp.sum(xr, axis=0, keepdims=True)
        r = 1.0 / (s + _EPS)
        xn = xr * r

        m = jnp.max(xr, axis=0, keepdims=True)
        inds = jnp.min(jnp.where(xr == m, iota, _DIM), axis=0, keepdims=True)

        one_hot = (iota == inds).astype(jnp.float32)
        quant_ref[k] = one_hot
        embed_ref[k] = inds

        hist_part += jnp.sum(one_hot, axis=1, keepdims=True)
        qsum_part += jnp.sum(xn, axis=1, keepdims=True)

    @pl.when(i == 0)
    def _init():
        hist_ref[...] = hist_part
        qsum_ref[...] = qsum_part

    @pl.when(i > 0)
    def _acc():
        hist_ref[...] += hist_part
        qsum_ref[...] += qsum_part

    @pl.when(i == nsteps - 1)
    def _finalize():
        n = float(_B * _HW)
        avg_probs = hist_ref[...] / n
        ent = jnp.sum(avg_probs * jnp.log(avg_probs + 1e-10), axis=0, keepdims=True)
        perp_ref[...] = jnp.exp(-ent)
        q_bar = qsum_ref[...] / n
        div_ref[...] = jnp.mean((q_bar * float(_DIM) - 1.0) ** 2, axis=0, keepdims=True)


def kernel(x):
    b, dim, h, w = x.shape
    hw = h * w
    xr = x.reshape(b, dim, hw)

    quant, embed, perp, div, _hist, _qsum = pl.pallas_call(
        _vq_kernel,
        grid=(b // _BB,),
        in_specs=[pl.BlockSpec((_BB, dim, hw), lambda i: (i, 0, 0))],
        out_specs=[
            pl.BlockSpec((_BB, dim, hw), lambda i: (i, 0, 0)),
            pl.BlockSpec((_BB, 1, hw), lambda i: (i, 0, 0)),
            pl.BlockSpec((1, 1), lambda i: (0, 0)),
            pl.BlockSpec((1, 1), lambda i: (0, 0)),
            pl.BlockSpec((dim, 1), lambda i: (0, 0)),
            pl.BlockSpec((dim, 1), lambda i: (0, 0)),
        ],
        out_shape=[
            jax.ShapeDtypeStruct((b, dim, hw), jnp.float32),
            jax.ShapeDtypeStruct((b, 1, hw), jnp.int32),
            jax.ShapeDtypeStruct((1, 1), jnp.float32),
            jax.ShapeDtypeStruct((1, 1), jnp.float32),
            jax.ShapeDtypeStruct((dim, 1), jnp.float32),
            jax.ShapeDtypeStruct((dim, 1), jnp.float32),
        ],
    )(xr)

    quantize = quant.reshape(b, dim, h, w)
    embed_ind = embed.reshape(b, h, w)
    return (quantize, div[0, 0], embed_ind, perp[0, 0])
